# Initial kernel scaffold; baseline (speedup 1.0000x reference)
import jax
import jax.numpy as jnp
from jax.experimental import pallas as pl


def kernel(target):
    raise NotImplementedError("write your pallas kernel here")


# speedup vs baseline: 1.0300x; 1.0300x over previous
import functools

import jax
import jax.numpy as jnp
from jax import lax
from jax.experimental import pallas as pl
from jax.experimental.pallas import tpu as pltpu
from jax.experimental.pallas import tpu_sc as plsc

_B = 16384
_C = 1000
_SMOOTH = 0.1
_NEG = _SMOOTH / _C
_POS = 1.0 - _SMOOTH + _NEG

_NC = 2
_NS = 16
_NW = _NC * _NS
_ROWS_W = _B // _NW
_CHUNK_ROWS = 64
_CHUNKS = _ROWS_W // _CHUNK_ROWS
_CHUNK_ELEMS = _CHUNK_ROWS * _C
_GROUPS = _CHUNK_ROWS // 16


def _body(target_hbm, out_hbm, tgt_v, buf0, buf1, sem0, sem1):
    wid = lax.axis_index("s") * _NC + lax.axis_index("c")
    rbase = wid * _ROWS_W
    pltpu.sync_copy(target_hbm.at[pl.ds(rbase, _ROWS_W)], tgt_v)

    neg16 = jnp.full((16,), _NEG, jnp.float32)
    pos16 = jnp.full((16,), _POS, jnp.float32)
    lane_row = lax.iota(jnp.int32, 16) * _C

    def fill(i, carry):
        base = i * 128
        for u in range(8):
            buf0[pl.ds(base + u * 16, 16)] = neg16
            buf1[pl.ds(base + u * 16, 16)] = neg16
        return carry

    lax.fori_loop(0, _CHUNK_ELEMS // 128, fill, 0)

    def scatter(c, buf, vec):
        for j in range(_GROUPS):
            t = tgt_v[pl.ds(c * _CHUNK_ROWS + j * 16, 16)]
            flat = lane_row + (j * 16 * _C) + t
            plsc.store_scatter(buf, [flat], vec)

    bufs = (buf0, buf1)
    sems = (sem0, sem1)
    copies = [None, None]
    for c in range(_CHUNKS):
        b = c % 2
        buf = bufs[b]
        if copies[b] is not None:
            copies[b].wait()
            scatter(c - 2, buf, neg16)
        scatter(c, buf, pos16)
        cp = pltpu.make_async_copy(
            buf,
            out_hbm.at[pl.ds((rbase + c * _CHUNK_ROWS) * _C, _CHUNK_ELEMS)],
            sems[b],
        )
        cp.start()
        copies[b] = cp
    copies[0].wait()
    copies[1].wait()


_sc_call = functools.partial(
    pl.kernel,
    out_type=jax.ShapeDtypeStruct((_B * _C,), jnp.float32),
    mesh=plsc.VectorSubcoreMesh(core_axis_name="c", subcore_axis_name="s"),
    compiler_params=pltpu.CompilerParams(needs_layout_passes=False),
    scratch_types=[
        pltpu.VMEM((_ROWS_W,), jnp.int32),
        pltpu.VMEM((_CHUNK_ELEMS,), jnp.float32),
        pltpu.VMEM((_CHUNK_ELEMS,), jnp.float32),
        pltpu.SemaphoreType.DMA,
        pltpu.SemaphoreType.DMA,
    ],
)(_body)


def kernel(target):
    return _sc_call(target).reshape(_B, _C)


# speedup vs baseline: 1.6897x; 1.6404x over previous
import functools

import jax
import jax.numpy as jnp
from jax import lax
from jax.experimental import pallas as pl
from jax.experimental.pallas import tpu as pltpu
from jax.experimental.pallas import tpu_sc as plsc

_B = 16384
_C = 1000
_SMOOTH = 0.1
_NEG = _SMOOTH / _C
_POS = 1.0 - _SMOOTH + _NEG

_NC = 2
_NS = 16
_NW = _NC * _NS
_ROWS_W = _B // _NW
_CHUNK_ROWS = 32
_CHUNKS = _ROWS_W // _CHUNK_ROWS
_GROUPS = _CHUNK_ROWS // 16
_COL_GROUPS = [k * 16 for k in range(_C // 16)] + [_C - 16]


def _body(target_hbm, out_hbm, tgt_v, buf0, buf1, sem0, sem1):
    wid = lax.axis_index("s") * _NC + lax.axis_index("c")
    rbase = wid * _ROWS_W
    pltpu.sync_copy(target_hbm.at[pl.ds(rbase, _ROWS_W)], tgt_v)

    neg16 = jnp.full((16,), _NEG, jnp.float32)
    pos16 = jnp.full((16,), _POS, jnp.float32)
    lane = lax.iota(jnp.int32, 16)

    def fill(r, carry):
        for cstart in _COL_GROUPS:
            buf0[r, pl.ds(cstart, 16)] = neg16
            buf1[r, pl.ds(cstart, 16)] = neg16
        return carry

    lax.fori_loop(0, _CHUNK_ROWS, fill, 0)

    def scatter(c, buf, vec):
        for j in range(_GROUPS):
            t = tgt_v[pl.ds(c * _CHUNK_ROWS + j * 16, 16)]
            plsc.store_scatter(buf, [lane + j * 16, t], vec)

    bufs = (buf0, buf1)
    sems = (sem0, sem1)
    copies = [None, None]
    for c in range(_CHUNKS):
        b = c % 2
        buf = bufs[b]
        if copies[b] is not None:
            copies[b].wait()
            scatter(c - 2, buf, neg16)
        scatter(c, buf, pos16)
        cp = pltpu.make_async_copy(
            buf,
            out_hbm.at[pl.ds(rbase + c * _CHUNK_ROWS, _CHUNK_ROWS)],
            sems[b],
        )
        cp.start()
        copies[b] = cp
    copies[0].wait()
    copies[1].wait()


_sc_call = functools.partial(
    pl.kernel,
    out_type=jax.ShapeDtypeStruct((_B, _C), jnp.float32),
    mesh=plsc.VectorSubcoreMesh(core_axis_name="c", subcore_axis_name="s"),
    compiler_params=pltpu.CompilerParams(needs_layout_passes=False),
    scratch_types=[
        pltpu.VMEM((_ROWS_W,), jnp.int32),
        pltpu.VMEM((_CHUNK_ROWS, _C), jnp.float32),
        pltpu.VMEM((_CHUNK_ROWS, _C), jnp.float32),
        pltpu.SemaphoreType.DMA,
        pltpu.SemaphoreType.DMA,
    ],
)(_body)


def kernel(target):
    return _sc_call(target)


# speedup vs baseline: 3.8424x; 2.2740x over previous
import functools

import jax
import jax.numpy as jnp
from jax import lax
from jax.experimental import pallas as pl
from jax.experimental.pallas import tpu as pltpu
from jax.experimental.pallas import tpu_sc as plsc

_B = 16384
_C = 1000
_SMOOTH = 0.1
_NEG = _SMOOTH / _C
_POS = 1.0 - _SMOOTH + _NEG

_NC = 2
_NS = 16
_NW = _NC * _NS
_BATCH_W = _B // _NW
_CHUNK_C = 40
_CHUNKS = _C // _CHUNK_C
_BGROUPS = _BATCH_W // 16


def _body(target_hbm, out_hbm, tgt_v, buf0, buf1, sem0, sem1):
    wid = lax.axis_index("s") * _NC + lax.axis_index("c")
    bbase = wid * _BATCH_W
    pltpu.sync_copy(target_hbm.at[pl.ds(bbase, _BATCH_W)], tgt_v)

    neg16 = jnp.full((16,), _NEG, jnp.float32)
    pos16 = jnp.full((16,), _POS, jnp.float32)
    lane = lax.iota(jnp.int32, 16)

    def fill(r, carry):
        for g in range(_BGROUPS):
            buf0[r, pl.ds(g * 16, 16)] = neg16
            buf1[r, pl.ds(g * 16, 16)] = neg16
        return carry

    lax.fori_loop(0, _CHUNK_C, fill, 0)

    def scatter(c, buf, vec):
        c0 = c * _CHUNK_C

        def one(g, carry):
            t = tgt_v[pl.ds(g * 16, 16)]
            m = (t >= c0) & (t < c0 + _CHUNK_C)
            plsc.store_scatter(buf, [t - c0, lane + g * 16], vec, mask=m)
            return carry

        lax.fori_loop(0, _BGROUPS, one, 0)

    bufs = (buf0, buf1)
    sems = (sem0, sem1)
    copies = [None, None]
    for c in range(_CHUNKS):
        b = c % 2
        buf = bufs[b]
        if copies[b] is not None:
            copies[b].wait()
            scatter(c - 2, buf, neg16)
        scatter(c, buf, pos16)
        cp = pltpu.make_async_copy(
            buf,
            out_hbm.at[pl.ds(c * _CHUNK_C, _CHUNK_C), pl.ds(bbase, _BATCH_W)],
            sems[b],
        )
        cp.start()
        copies[b] = cp
    copies[0].wait()
    copies[1].wait()


_sc_call = functools.partial(
    pl.kernel,
    out_type=jax.ShapeDtypeStruct((_C, _B), jnp.float32),
    mesh=plsc.VectorSubcoreMesh(core_axis_name="c", subcore_axis_name="s"),
    compiler_params=pltpu.CompilerParams(needs_layout_passes=False),
    scratch_types=[
        pltpu.VMEM((_BATCH_W,), jnp.int32),
        pltpu.VMEM((_CHUNK_C, _BATCH_W), jnp.float32),
        pltpu.VMEM((_CHUNK_C, _BATCH_W), jnp.float32),
        pltpu.SemaphoreType.DMA,
        pltpu.SemaphoreType.DMA,
    ],
)(_body)


def kernel(target):
    return _sc_call(target).T
